# baseline (speedup 1.0000x reference)
import functools
import math

import jax
import jax.numpy as jnp
from jax import lax
from jax.experimental import pallas as pl
from jax.experimental.pallas import tpu as pltpu
from jax.experimental.pallas import tpu_sc as plsc

EMB = 64
SCALE = math.sqrt(EMB)
C = 128
LANES = 16


def _emb_lookup(tok, table, *, nc, ns):
    nw = nc * ns
    chunks = tok.shape[1]
    b_per_w = chunks * C
    b_total = nw * b_per_w
    mesh = plsc.VectorSubcoreMesh(core_axis_name="c", subcore_axis_name="s")

    @functools.partial(
        pl.kernel,
        out_type=jax.ShapeDtypeStruct((b_total, EMB), jnp.float32),
        mesh=mesh,
        scratch_types=[
            pltpu.VMEM((chunks, C), jnp.int32),
            pltpu.VMEM((C, EMB), jnp.float32),
            pltpu.VMEM((C, EMB), jnp.float32),
            pltpu.SemaphoreType.DMA,
            pltpu.SemaphoreType.DMA,
        ],
        compiler_params=pltpu.CompilerParams(use_tc_tiling_on_sc=False),
    )
    def run(tok_hbm, table_hbm, out_hbm, idx_v, buf0, buf1, sem0, sem1):
        wid = lax.axis_index("s") * nc + lax.axis_index("c")
        base = wid * b_per_w
        bufs = (buf0, buf1)
        sems = (sem0, sem1)

        pltpu.sync_copy(tok_hbm.at[wid], idx_v)

        pltpu.async_copy(table_hbm.at[idx_v.at[0]], buf0, sem0)
        pltpu.async_copy(table_hbm.at[idx_v.at[1]], buf1, sem1)

        def scale_rows(buf):
            def row(r, carry):
                for k in range(EMB // LANES):
                    sl = pl.ds(k * LANES, LANES)
                    buf[r, sl] = buf[r, sl] * SCALE
                return carry

            lax.fori_loop(0, C, row, 0)

        def do_chunk(g, b, *, start_next):
            pltpu.make_async_copy(
                table_hbm.at[pl.ds(0, C)], bufs[b], sems[b]
            ).wait()
            scale_rows(bufs[b])
            pltpu.sync_copy(bufs[b], out_hbm.at[pl.ds(base + g * C, C)])
            if start_next:
                pltpu.async_copy(table_hbm.at[idx_v.at[g + 2]], bufs[b], sems[b])

        def step(g2, carry):
            for b in range(2):
                do_chunk(g2 * 2 + b, b, start_next=True)
            return carry

        lax.fori_loop(0, chunks // 2 - 1, step, 0)
        for b in range(2):
            do_chunk(chunks - 2 + b, b, start_next=False)

    return run(tok, table)


def kernel(tokens, table):
    b0, s = tokens.shape
    b_total = b0 * s
    info = plsc.get_sparse_core_info()
    nc, ns = info.num_cores, info.num_subcores
    nw = nc * ns
    b_per_w = b_total // nw
    chunks = b_per_w // C
    tok = tokens.astype(jnp.int32).reshape(nw, chunks, C)
    out = _emb_lookup(tok, table, nc=nc, ns=ns)
    return out.reshape(b0, s, EMB)
